# baseline (speedup 1.0000x reference)
import functools

import jax
import jax.numpy as jnp
from jax import lax
from jax.experimental import pallas as pl
from jax.experimental.pallas import tpu as pltpu
from jax.experimental.pallas import tpu_sc as plsc

_NUM_CORES = 2
_NUM_SUBCORES = 16
_NUM_WORKERS = _NUM_CORES * _NUM_SUBCORES


def kernel(data, partitions):
    n_rows, n_cols = data.shape
    rows_per_w = n_rows // _NUM_WORKERS
    mesh = plsc.VectorSubcoreMesh(
        core_axis_name="c", subcore_axis_name="s",
        num_cores=_NUM_CORES, num_subcores=_NUM_SUBCORES)

    @functools.partial(
        pl.kernel,
        mesh=mesh,
        out_type=jax.ShapeDtypeStruct((n_rows, n_cols), data.dtype),
    )
    def run(data_hbm, part_hbm, out_hbm):
        del part_hbm
        wid = lax.axis_index("s") * _NUM_CORES + lax.axis_index("c")
        base = wid * rows_per_w
        pltpu.sync_copy(data_hbm.at[pl.ds(base, rows_per_w)],
                        out_hbm.at[pl.ds(base, rows_per_w)])

    return run(data, partitions)


# speedup vs baseline: 35.0040x; 35.0040x over previous
import functools

import jax
import jax.numpy as jnp
from jax import lax
from jax.experimental import pallas as pl
from jax.experimental.pallas import tpu as pltpu
from jax.experimental.pallas import tpu_sc as plsc

_NUM_CORES = 2
_NUM_SUBCORES = 16
_NUM_WORKERS = _NUM_CORES * _NUM_SUBCORES
_CHUNK = 32
_NBUF = 2


def kernel(data, partitions):
    n_rows, n_cols = data.shape
    rows_per_w = n_rows // _NUM_WORKERS
    n_chunks = rows_per_w // _CHUNK
    n_groups = n_chunks // _NBUF
    mesh = plsc.VectorSubcoreMesh(
        core_axis_name="c", subcore_axis_name="s",
        num_cores=_NUM_CORES, num_subcores=_NUM_SUBCORES)

    @functools.partial(
        pl.kernel,
        mesh=mesh,
        out_type=jax.ShapeDtypeStruct((n_rows, n_cols), data.dtype),
        scratch_types=[
            pltpu.VMEM((_NBUF, _CHUNK, n_cols), jnp.float32),
            pltpu.SemaphoreType.DMA,
            pltpu.SemaphoreType.DMA,
            pltpu.SemaphoreType.DMA,
            pltpu.SemaphoreType.DMA,
        ],
    )
    def run(data_hbm, part_hbm, out_hbm, buf, sin0, sin1, sout0, sout1):
        del part_hbm
        sin = (sin0, sin1)
        sout = (sout0, sout1)
        wid = lax.axis_index("s") * _NUM_CORES + lax.axis_index("c")
        base = wid * rows_per_w

        def grp(g, carry):
            for b in range(_NBUF):
                off = base + (g * _NBUF + b) * _CHUNK
                prev_off = base + ((g - 1) * _NBUF + b) * _CHUNK

                @pl.when(g > 0)
                def _():
                    pltpu.make_async_copy(
                        buf.at[b], out_hbm.at[pl.ds(prev_off, _CHUNK)],
                        sout[b]).wait()

                pltpu.async_copy(
                    data_hbm.at[pl.ds(off, _CHUNK)], buf.at[b], sin[b])
            for b in range(_NBUF):
                off = base + (g * _NBUF + b) * _CHUNK
                pltpu.make_async_copy(
                    data_hbm.at[pl.ds(off, _CHUNK)], buf.at[b], sin[b]).wait()
                pltpu.async_copy(
                    buf.at[b], out_hbm.at[pl.ds(off, _CHUNK)], sout[b])
            return carry

        lax.fori_loop(0, n_groups, grp, 0)
        for b in range(_NBUF):
            off = base + ((n_groups - 1) * _NBUF + b) * _CHUNK
            pltpu.make_async_copy(
                buf.at[b], out_hbm.at[pl.ds(off, _CHUNK)], sout[b]).wait()

    return run(data, partitions)
